# baseline (speedup 1.0000x reference)
import functools

import jax
import jax.numpy as jnp
from jax import lax
from jax.experimental import pallas as pl
from jax.experimental.pallas import tpu as pltpu
from jax.experimental.pallas import tpu_sc as plsc

NC = 2
NS = 16
NW = NC * NS

CHUNK = 128
D = 128

N_NODES = 10000
ACC_ROWS = 10240
SLT = ACC_ROWS // NS
NPIECE = SLT // CHUNK
DUMMY = N_NODES

E_EDGES = 320000

CPT0 = 107
CPT1 = 50
CPTM = CPT0
E0 = NS * CPT0 * CHUNK
E1 = E_EDGES - E0
E1_SLOTS = NS * CPT1 * CHUNK

DCPT = 80
DE_PAD = NW * DCPT * CHUNK
DBATCH = 16


def _seg_body(row_hbm, col_hbm, h_hbm, out_hbm,
              idx_r, idx_c, rows_v, acc, sem):
  cid = lax.axis_index("c")
  sid = lax.axis_index("s")
  wid = cid * NS + sid
  base = sid * SLT

  def zfill(i, carry):
    def zlane(j, c2):
      rows_v[i, pl.ds(j * 16, 16)] = jnp.zeros((16,), jnp.float32)
      return c2
    return lax.fori_loop(0, D // 16, zlane, carry)
  lax.fori_loop(0, CHUNK, zfill, 0)
  for p in range(NPIECE):
    pltpu.sync_copy(rows_v, acc.at[pl.ds(base + p * CHUNK, CHUNK)])
  plsc.subcore_barrier()

  def chunk_step(c, carry):
    pltpu.sync_copy(row_hbm.at[wid, c], idx_r)
    pltpu.sync_copy(col_hbm.at[wid, c], idx_c)
    pltpu.async_copy(h_hbm.at[idx_r], rows_v, sem).wait()
    pltpu.sync_copy(rows_v, acc.at[idx_c], add=True)
    return carry

  n_chunks = jnp.where(cid == 0, CPT0, CPT1)
  lax.fori_loop(0, n_chunks, chunk_step, 0)
  plsc.subcore_barrier()

  obase = cid * ACC_ROWS + base
  for p in range(NPIECE):
    pltpu.sync_copy(acc.at[pl.ds(base + p * CHUNK, CHUNK)], rows_v)
    pltpu.sync_copy(rows_v, out_hbm.at[pl.ds(obase + p * CHUNK, CHUNK)])


@functools.lru_cache(maxsize=None)
def _seg_sum_kernel():
  mesh = plsc.VectorSubcoreMesh(
      core_axis_name="c", subcore_axis_name="s",
      num_cores=NC, num_subcores=NS)
  return pl.kernel(
      _seg_body, mesh=mesh,
      out_type=jax.ShapeDtypeStruct((NC * ACC_ROWS, D), jnp.float32),
      scratch_types=[
          pltpu.VMEM((CHUNK,), jnp.int32),
          pltpu.VMEM((CHUNK,), jnp.int32),
          pltpu.VMEM((CHUNK, D), jnp.float32),
          pltpu.VMEM_SHARED((ACC_ROWS, D), jnp.float32),
          pltpu.SemaphoreType.DMA,
      ],
  )


def _pad_dummy(n, offset=0):
  return (jnp.arange(offset, offset + n, dtype=jnp.int32)
          % (ACC_ROWS - N_NODES)) + DUMMY


def _deg_body(col_hbm, out_hbm, coli, ones_v, stage_v, acc, sem):
  cid = lax.axis_index("c")
  sid = lax.axis_index("s")
  wid = cid * NS + sid

  pltpu.sync_copy(col_hbm.at[wid], coli)

  for i in range(CHUNK // 16):
    ones_v[pl.ds(i * 16, 16)] = jnp.full((16,), 1.0, jnp.float32)

  def zfill(i, carry):
    stage_v[pl.ds(i * 16, 16)] = jnp.zeros((16,), jnp.float32)
    return carry
  lax.fori_loop(0, SLT // 16, zfill, 0)
  pltpu.sync_copy(stage_v, acc.at[pl.ds(sid * SLT, SLT)])
  plsc.subcore_barrier()

  def batch_step(bt, carry):
    base = bt * DBATCH
    def fire(k, c2):
      pltpu.async_copy(ones_v, acc.at[coli.at[base + k]], sem, add=True)
      return c2
    lax.fori_loop(0, DBATCH, fire, 0)
    def drain(k, c2):
      pltpu.make_async_copy(ones_v, acc.at[coli.at[base + k]], sem).wait()
      return c2
    lax.fori_loop(0, DBATCH, drain, 0)
    return carry

  lax.fori_loop(0, DCPT // DBATCH, batch_step, 0)
  plsc.subcore_barrier()

  pltpu.sync_copy(acc.at[pl.ds(sid * SLT, SLT)], stage_v)
  pltpu.sync_copy(stage_v,
                  out_hbm.at[pl.ds(cid * ACC_ROWS + sid * SLT, SLT)])


@functools.lru_cache(maxsize=None)
def _deg_sum_kernel():
  mesh = plsc.VectorSubcoreMesh(
      core_axis_name="c", subcore_axis_name="s",
      num_cores=NC, num_subcores=NS)
  return pl.kernel(
      _deg_body, mesh=mesh,
      out_type=jax.ShapeDtypeStruct((NC * ACC_ROWS,), jnp.float32),
      scratch_types=[
          pltpu.VMEM((DCPT, CHUNK), jnp.int32),
          pltpu.VMEM((CHUNK,), jnp.float32),
          pltpu.VMEM((SLT,), jnp.float32),
          pltpu.VMEM_SHARED((ACC_ROWS,), jnp.float32),
          pltpu.SemaphoreType.DMA,
      ],
  )


ROW_BLK = 1000
GRID = N_NODES // ROW_BLK


def _stage_a_body(x_ref, w_ref, d0_ref, d1_ref, hp_ref, dis_ref):
  dis = lax.rsqrt(d0_ref[...] + d1_ref[...] + 1.0)
  h = jnp.dot(x_ref[...], w_ref[...], preferred_element_type=jnp.float32)
  hp_ref[...] = dis * h
  dis_ref[...] = dis


def _stage_a(x, w1, d0, d1):
  return pl.pallas_call(
      _stage_a_body,
      grid=(GRID,),
      in_specs=[
          pl.BlockSpec((ROW_BLK, D), lambda i: (i, 0)),
          pl.BlockSpec((D, D), lambda i: (0, 0)),
          pl.BlockSpec((ROW_BLK, 1), lambda i: (i, 0)),
          pl.BlockSpec((ROW_BLK, 1), lambda i: (i, 0)),
      ],
      out_specs=[
          pl.BlockSpec((ROW_BLK, D), lambda i: (i, 0)),
          pl.BlockSpec((ROW_BLK, 1), lambda i: (i, 0)),
      ],
      out_shape=[
          jax.ShapeDtypeStruct((N_NODES, D), jnp.float32),
          jax.ShapeDtypeStruct((N_NODES, 1), jnp.float32),
      ],
  )(x, w1, d0, d1)


def _stage_b_body(p0_ref, p1_ref, hp_ref, dis_ref, b_ref, w_ref, out_ref):
  dis = dis_ref[...]
  a = dis * (p0_ref[...] + p1_ref[...] + hp_ref[...]) + b_ref[...]
  a = jnp.maximum(a, 0.0)
  out_ref[...] = dis * jnp.dot(a, w_ref[...],
                               preferred_element_type=jnp.float32)


def _stage_b(p0, p1, hp, dis, b1, w2):
  return pl.pallas_call(
      _stage_b_body,
      grid=(GRID,),
      in_specs=[
          pl.BlockSpec((ROW_BLK, D), lambda i: (i, 0)),
          pl.BlockSpec((ROW_BLK, D), lambda i: (i, 0)),
          pl.BlockSpec((ROW_BLK, D), lambda i: (i, 0)),
          pl.BlockSpec((ROW_BLK, 1), lambda i: (i, 0)),
          pl.BlockSpec((1, D), lambda i: (0, 0)),
          pl.BlockSpec((D, D), lambda i: (0, 0)),
      ],
      out_specs=pl.BlockSpec((ROW_BLK, D), lambda i: (i, 0)),
      out_shape=jax.ShapeDtypeStruct((N_NODES, D), jnp.float32),
  )(p0, p1, hp, dis, b1, w2)


def _stage_c_body(p0_ref, p1_ref, hp_ref, dis_ref, b_ref, wfc_ref, bfc_ref,
                  out_ref, acc_ref):
  i = pl.program_id(0)

  @pl.when(i == 0)
  def _():
    acc_ref[...] = jnp.zeros_like(acc_ref)

  a = dis_ref[...] * (p0_ref[...] + p1_ref[...] + hp_ref[...]) + b_ref[...]
  acc_ref[...] += jnp.sum(a, axis=0, keepdims=True)

  @pl.when(i == GRID - 1)
  def _():
    g = acc_ref[...] * (1.0 / N_NODES)
    out_ref[...] = lax.dot_general(
        g, wfc_ref[...], (((1,), (1,)), ((), ())),
        preferred_element_type=jnp.float32) + bfc_ref[...]


def _stage_c(p0, p1, hp, dis, b2, wfc, bfc):
  return pl.pallas_call(
      _stage_c_body,
      grid=(GRID,),
      in_specs=[
          pl.BlockSpec((ROW_BLK, D), lambda i: (i, 0)),
          pl.BlockSpec((ROW_BLK, D), lambda i: (i, 0)),
          pl.BlockSpec((ROW_BLK, D), lambda i: (i, 0)),
          pl.BlockSpec((ROW_BLK, 1), lambda i: (i, 0)),
          pl.BlockSpec((1, D), lambda i: (0, 0)),
          pl.BlockSpec((40, D), lambda i: (0, 0)),
          pl.BlockSpec((1, 40), lambda i: (0, 0)),
      ],
      out_specs=pl.BlockSpec((1, 40), lambda i: (0, 0)),
      out_shape=jax.ShapeDtypeStruct((1, 40), jnp.float32),
      scratch_shapes=[pltpu.VMEM((1, D), jnp.float32)],
  )(p0, p1, hp, dis, b2, wfc, bfc)


def _halves(s):
  return s[:N_NODES], s[ACC_ROWS:ACC_ROWS + N_NODES]


def kernel(x, edge_index, W1, b1, W2, b2, Wfc, bfc):
  row = edge_index[0]
  col = edge_index[1]

  pad1 = E1_SLOTS - E1
  row_sc0 = row[:E0].reshape(NS, CPT0, CHUNK)
  col_sc0 = col[:E0].reshape(NS, CPT0, CHUNK)
  row_sc1 = jnp.concatenate([row[E0:], jnp.zeros((pad1,), jnp.int32)])
  col_sc1 = jnp.concatenate([col[E0:], _pad_dummy(pad1)])
  row_sc1 = row_sc1.reshape(NS, CPT1, CHUNK)
  col_sc1 = col_sc1.reshape(NS, CPT1, CHUNK)
  fill = ((0, 0), (0, CPTM - CPT1), (0, 0))
  row3 = jnp.concatenate(
      [row_sc0, jnp.pad(row_sc1, fill)], axis=0)
  col3 = jnp.concatenate(
      [col_sc0, jnp.pad(col_sc1, fill, constant_values=DUMMY)], axis=0)

  pad_d = DE_PAD - E_EDGES
  col3d = jnp.concatenate([col, _pad_dummy(pad_d)]).reshape(NW, DCPT, CHUNK)

  degp = _deg_sum_kernel()(col3d)
  degp = degp.reshape(-1, 1)
  d0, d1 = _halves(degp)

  hp1, dis = _stage_a(x, W1, d0, d1)
  s1a, s1b = _halves(_seg_sum_kernel()(row3, col3, hp1))
  hp2 = _stage_b(s1a, s1b, hp1, dis, b1.reshape(1, D), W2)
  s2a, s2b = _halves(_seg_sum_kernel()(row3, col3, hp2))
  out = _stage_c(s2a, s2b, hp2, dis, b2.reshape(1, D), Wfc,
                 bfc.reshape(1, 40))
  return out


# speedup vs baseline: 1.0039x; 1.0039x over previous
import functools

import jax
import jax.numpy as jnp
from jax import lax
from jax.experimental import pallas as pl
from jax.experimental.pallas import tpu as pltpu
from jax.experimental.pallas import tpu_sc as plsc

NC = 2
NS = 16
NW = NC * NS

CHUNK = 128
D = 128

N_NODES = 10000
ACC_ROWS = 10240
SLT = ACC_ROWS // NS
NPIECE = SLT // CHUNK
DUMMY = N_NODES

E_EDGES = 320000

CPT0 = 106
CPT1 = 51
CPTM = CPT0
E0 = NS * CPT0 * CHUNK
E1 = E_EDGES - E0
E1_SLOTS = NS * CPT1 * CHUNK

DCPT = 80
DE_PAD = NW * DCPT * CHUNK
DBATCH = 16


def _seg_body(row_hbm, col_hbm, h_hbm, out_hbm,
              idx_r, idx_c, rows_v, acc, sem):
  cid = lax.axis_index("c")
  sid = lax.axis_index("s")
  wid = cid * NS + sid
  base = sid * SLT

  def zfill(i, carry):
    def zlane(j, c2):
      rows_v[i, pl.ds(j * 16, 16)] = jnp.zeros((16,), jnp.float32)
      return c2
    return lax.fori_loop(0, D // 16, zlane, carry)
  lax.fori_loop(0, CHUNK, zfill, 0)
  for p in range(NPIECE):
    pltpu.sync_copy(rows_v, acc.at[pl.ds(base + p * CHUNK, CHUNK)])
  plsc.subcore_barrier()

  def chunk_step(c, carry):
    pltpu.sync_copy(row_hbm.at[wid, c], idx_r)
    pltpu.sync_copy(col_hbm.at[wid, c], idx_c)
    pltpu.async_copy(h_hbm.at[idx_r], rows_v, sem).wait()
    pltpu.sync_copy(rows_v, acc.at[idx_c], add=True)
    return carry

  n_chunks = jnp.where(cid == 0, CPT0, CPT1)
  lax.fori_loop(0, n_chunks, chunk_step, 0)
  plsc.subcore_barrier()

  obase = cid * ACC_ROWS + base
  for p in range(NPIECE):
    pltpu.sync_copy(acc.at[pl.ds(base + p * CHUNK, CHUNK)], rows_v)
    pltpu.sync_copy(rows_v, out_hbm.at[pl.ds(obase + p * CHUNK, CHUNK)])


@functools.lru_cache(maxsize=None)
def _seg_sum_kernel():
  mesh = plsc.VectorSubcoreMesh(
      core_axis_name="c", subcore_axis_name="s",
      num_cores=NC, num_subcores=NS)
  return pl.kernel(
      _seg_body, mesh=mesh,
      out_type=jax.ShapeDtypeStruct((NC * ACC_ROWS, D), jnp.float32),
      scratch_types=[
          pltpu.VMEM((CHUNK,), jnp.int32),
          pltpu.VMEM((CHUNK,), jnp.int32),
          pltpu.VMEM((CHUNK, D), jnp.float32),
          pltpu.VMEM_SHARED((ACC_ROWS, D), jnp.float32),
          pltpu.SemaphoreType.DMA,
      ],
  )


def _pad_dummy(n, offset=0):
  return (jnp.arange(offset, offset + n, dtype=jnp.int32)
          % (ACC_ROWS - N_NODES)) + DUMMY


def _deg_body(col_hbm, out_hbm, coli, ones_v, stage_v, acc, sem):
  cid = lax.axis_index("c")
  sid = lax.axis_index("s")
  wid = cid * NS + sid

  pltpu.sync_copy(col_hbm.at[wid], coli)

  for i in range(CHUNK // 16):
    ones_v[pl.ds(i * 16, 16)] = jnp.full((16,), 1.0, jnp.float32)

  def zfill(i, carry):
    stage_v[pl.ds(i * 16, 16)] = jnp.zeros((16,), jnp.float32)
    return carry
  lax.fori_loop(0, SLT // 16, zfill, 0)
  pltpu.sync_copy(stage_v, acc.at[pl.ds(sid * SLT, SLT)])
  plsc.subcore_barrier()

  def batch_step(bt, carry):
    base = bt * DBATCH
    def fire(k, c2):
      pltpu.async_copy(ones_v, acc.at[coli.at[base + k]], sem, add=True)
      return c2
    lax.fori_loop(0, DBATCH, fire, 0)
    def drain(k, c2):
      pltpu.make_async_copy(ones_v, acc.at[coli.at[base + k]], sem).wait()
      return c2
    lax.fori_loop(0, DBATCH, drain, 0)
    return carry

  lax.fori_loop(0, DCPT // DBATCH, batch_step, 0)
  plsc.subcore_barrier()

  pltpu.sync_copy(acc.at[pl.ds(sid * SLT, SLT)], stage_v)
  pltpu.sync_copy(stage_v,
                  out_hbm.at[pl.ds(cid * ACC_ROWS + sid * SLT, SLT)])


@functools.lru_cache(maxsize=None)
def _deg_sum_kernel():
  mesh = plsc.VectorSubcoreMesh(
      core_axis_name="c", subcore_axis_name="s",
      num_cores=NC, num_subcores=NS)
  return pl.kernel(
      _deg_body, mesh=mesh,
      out_type=jax.ShapeDtypeStruct((NC * ACC_ROWS,), jnp.float32),
      scratch_types=[
          pltpu.VMEM((DCPT, CHUNK), jnp.int32),
          pltpu.VMEM((CHUNK,), jnp.float32),
          pltpu.VMEM((SLT,), jnp.float32),
          pltpu.VMEM_SHARED((ACC_ROWS,), jnp.float32),
          pltpu.SemaphoreType.DMA,
      ],
  )


ROW_BLK = 1000
GRID = N_NODES // ROW_BLK


def _stage_a_body(x_ref, w_ref, d0_ref, d1_ref, hp_ref, dis_ref):
  dis = lax.rsqrt(d0_ref[...] + d1_ref[...] + 1.0)
  h = jnp.dot(x_ref[...], w_ref[...], preferred_element_type=jnp.float32)
  hp_ref[...] = dis * h
  dis_ref[...] = dis


def _stage_a(x, w1, d0, d1):
  return pl.pallas_call(
      _stage_a_body,
      grid=(GRID,),
      in_specs=[
          pl.BlockSpec((ROW_BLK, D), lambda i: (i, 0)),
          pl.BlockSpec((D, D), lambda i: (0, 0)),
          pl.BlockSpec((ROW_BLK, 1), lambda i: (i, 0)),
          pl.BlockSpec((ROW_BLK, 1), lambda i: (i, 0)),
      ],
      out_specs=[
          pl.BlockSpec((ROW_BLK, D), lambda i: (i, 0)),
          pl.BlockSpec((ROW_BLK, 1), lambda i: (i, 0)),
      ],
      out_shape=[
          jax.ShapeDtypeStruct((N_NODES, D), jnp.float32),
          jax.ShapeDtypeStruct((N_NODES, 1), jnp.float32),
      ],
  )(x, w1, d0, d1)


def _stage_b_body(p0_ref, p1_ref, hp_ref, dis_ref, b_ref, w_ref, out_ref):
  dis = dis_ref[...]
  a = dis * (p0_ref[...] + p1_ref[...] + hp_ref[...]) + b_ref[...]
  a = jnp.maximum(a, 0.0)
  out_ref[...] = dis * jnp.dot(a, w_ref[...],
                               preferred_element_type=jnp.float32)


def _stage_b(p0, p1, hp, dis, b1, w2):
  return pl.pallas_call(
      _stage_b_body,
      grid=(GRID,),
      in_specs=[
          pl.BlockSpec((ROW_BLK, D), lambda i: (i, 0)),
          pl.BlockSpec((ROW_BLK, D), lambda i: (i, 0)),
          pl.BlockSpec((ROW_BLK, D), lambda i: (i, 0)),
          pl.BlockSpec((ROW_BLK, 1), lambda i: (i, 0)),
          pl.BlockSpec((1, D), lambda i: (0, 0)),
          pl.BlockSpec((D, D), lambda i: (0, 0)),
      ],
      out_specs=pl.BlockSpec((ROW_BLK, D), lambda i: (i, 0)),
      out_shape=jax.ShapeDtypeStruct((N_NODES, D), jnp.float32),
  )(p0, p1, hp, dis, b1, w2)


def _stage_c_body(p0_ref, p1_ref, hp_ref, dis_ref, b_ref, wfc_ref, bfc_ref,
                  out_ref, acc_ref):
  i = pl.program_id(0)

  @pl.when(i == 0)
  def _():
    acc_ref[...] = jnp.zeros_like(acc_ref)

  a = dis_ref[...] * (p0_ref[...] + p1_ref[...] + hp_ref[...]) + b_ref[...]
  acc_ref[...] += jnp.sum(a, axis=0, keepdims=True)

  @pl.when(i == GRID - 1)
  def _():
    g = acc_ref[...] * (1.0 / N_NODES)
    out_ref[...] = lax.dot_general(
        g, wfc_ref[...], (((1,), (1,)), ((), ())),
        preferred_element_type=jnp.float32) + bfc_ref[...]


def _stage_c(p0, p1, hp, dis, b2, wfc, bfc):
  return pl.pallas_call(
      _stage_c_body,
      grid=(GRID,),
      in_specs=[
          pl.BlockSpec((ROW_BLK, D), lambda i: (i, 0)),
          pl.BlockSpec((ROW_BLK, D), lambda i: (i, 0)),
          pl.BlockSpec((ROW_BLK, D), lambda i: (i, 0)),
          pl.BlockSpec((ROW_BLK, 1), lambda i: (i, 0)),
          pl.BlockSpec((1, D), lambda i: (0, 0)),
          pl.BlockSpec((40, D), lambda i: (0, 0)),
          pl.BlockSpec((1, 40), lambda i: (0, 0)),
      ],
      out_specs=pl.BlockSpec((1, 40), lambda i: (0, 0)),
      out_shape=jax.ShapeDtypeStruct((1, 40), jnp.float32),
      scratch_shapes=[pltpu.VMEM((1, D), jnp.float32)],
  )(p0, p1, hp, dis, b2, wfc, bfc)


def _halves(s):
  return s[:N_NODES], s[ACC_ROWS:ACC_ROWS + N_NODES]


def kernel(x, edge_index, W1, b1, W2, b2, Wfc, bfc):
  row = edge_index[0]
  col = edge_index[1]

  pad1 = E1_SLOTS - E1
  row_sc0 = row[:E0].reshape(NS, CPT0, CHUNK)
  col_sc0 = col[:E0].reshape(NS, CPT0, CHUNK)
  row_sc1 = jnp.concatenate([row[E0:], jnp.zeros((pad1,), jnp.int32)])
  col_sc1 = jnp.concatenate([col[E0:], _pad_dummy(pad1)])
  row_sc1 = row_sc1.reshape(NS, CPT1, CHUNK)
  col_sc1 = col_sc1.reshape(NS, CPT1, CHUNK)
  fill = ((0, 0), (0, CPTM - CPT1), (0, 0))
  row3 = jnp.concatenate(
      [row_sc0, jnp.pad(row_sc1, fill)], axis=0)
  col3 = jnp.concatenate(
      [col_sc0, jnp.pad(col_sc1, fill, constant_values=DUMMY)], axis=0)

  pad_d = DE_PAD - E_EDGES
  col3d = jnp.concatenate([col, _pad_dummy(pad_d)]).reshape(NW, DCPT, CHUNK)

  degp = _deg_sum_kernel()(col3d)
  degp = degp.reshape(-1, 1)
  d0, d1 = _halves(degp)

  hp1, dis = _stage_a(x, W1, d0, d1)
  s1a, s1b = _halves(_seg_sum_kernel()(row3, col3, hp1))
  hp2 = _stage_b(s1a, s1b, hp1, dis, b1.reshape(1, D), W2)
  s2a, s2b = _halves(_seg_sum_kernel()(row3, col3, hp2))
  out = _stage_c(s2a, s2b, hp2, dis, b2.reshape(1, D), Wfc,
                 bfc.reshape(1, 40))
  return out


# speedup vs baseline: 1.0084x; 1.0044x over previous
import functools

import jax
import jax.numpy as jnp
from jax import lax
from jax.experimental import pallas as pl
from jax.experimental.pallas import tpu as pltpu
from jax.experimental.pallas import tpu_sc as plsc

NC = 2
NS = 16
NW = NC * NS

CHUNK = 128
D = 128

N_NODES = 10000
ACC_ROWS = 10240
SLT = ACC_ROWS // NS
NPIECE = SLT // CHUNK
DUMMY = N_NODES

E_EDGES = 320000

CPT0 = 106
CPT1 = 51
CPTM = CPT0
E0 = NS * CPT0 * CHUNK
E1 = E_EDGES - E0
E1_SLOTS = NS * CPT1 * CHUNK

DCPT = 80
DE_PAD = NW * DCPT * CHUNK
DBATCH = 16


def _seg_body(row_hbm, col_hbm, h_hbm, out_hbm,
              idx_r, idx_c, rows_v, acc, sem):
  cid = lax.axis_index("c")
  sid = lax.axis_index("s")
  wid = cid * NS + sid
  base = sid * SLT

  def zfill(i, carry):
    def zlane(j, c2):
      rows_v[i, pl.ds(j * 16, 16)] = jnp.zeros((16,), jnp.float32)
      return c2
    return lax.fori_loop(0, D // 16, zlane, carry)
  lax.fori_loop(0, CHUNK, zfill, 0)
  for p in range(NPIECE):
    pltpu.sync_copy(rows_v, acc.at[pl.ds(base + p * CHUNK, CHUNK)])
  plsc.subcore_barrier()

  def chunk_step(c, carry):
    pltpu.sync_copy(row_hbm.at[wid, c], idx_r)
    pltpu.sync_copy(col_hbm.at[wid, c], idx_c)
    pltpu.async_copy(h_hbm.at[idx_r], rows_v, sem).wait()
    pltpu.sync_copy(rows_v, acc.at[idx_c], add=True)
    return carry

  n_chunks = jnp.where(cid == 0, CPT0, CPT1)
  lax.fori_loop(0, n_chunks, chunk_step, 0)
  plsc.subcore_barrier()

  obase = cid * ACC_ROWS + base
  pltpu.sync_copy(acc.at[pl.ds(base, SLT)], out_hbm.at[pl.ds(obase, SLT)])


@functools.lru_cache(maxsize=None)
def _seg_sum_kernel():
  mesh = plsc.VectorSubcoreMesh(
      core_axis_name="c", subcore_axis_name="s",
      num_cores=NC, num_subcores=NS)
  return pl.kernel(
      _seg_body, mesh=mesh,
      out_type=jax.ShapeDtypeStruct((NC * ACC_ROWS, D), jnp.float32),
      scratch_types=[
          pltpu.VMEM((CHUNK,), jnp.int32),
          pltpu.VMEM((CHUNK,), jnp.int32),
          pltpu.VMEM((CHUNK, D), jnp.float32),
          pltpu.VMEM_SHARED((ACC_ROWS, D), jnp.float32),
          pltpu.SemaphoreType.DMA,
      ],
  )


def _pad_dummy(n, offset=0):
  return (jnp.arange(offset, offset + n, dtype=jnp.int32)
          % (ACC_ROWS - N_NODES)) + DUMMY


def _deg_body(col_hbm, out_hbm, coli, ones_v, stage_v, acc, sem):
  cid = lax.axis_index("c")
  sid = lax.axis_index("s")
  wid = cid * NS + sid

  pltpu.sync_copy(col_hbm.at[wid], coli)

  for i in range(CHUNK // 16):
    ones_v[pl.ds(i * 16, 16)] = jnp.full((16,), 1.0, jnp.float32)

  def zfill(i, carry):
    stage_v[pl.ds(i * 16, 16)] = jnp.zeros((16,), jnp.float32)
    return carry
  lax.fori_loop(0, SLT // 16, zfill, 0)
  pltpu.sync_copy(stage_v, acc.at[pl.ds(sid * SLT, SLT)])
  plsc.subcore_barrier()

  def batch_step(bt, carry):
    base = bt * DBATCH
    def fire(k, c2):
      pltpu.async_copy(ones_v, acc.at[coli.at[base + k]], sem, add=True)
      return c2
    lax.fori_loop(0, DBATCH, fire, 0)
    def drain(k, c2):
      pltpu.make_async_copy(ones_v, acc.at[coli.at[base + k]], sem).wait()
      return c2
    lax.fori_loop(0, DBATCH, drain, 0)
    return carry

  lax.fori_loop(0, DCPT // DBATCH, batch_step, 0)
  plsc.subcore_barrier()

  pltpu.sync_copy(acc.at[pl.ds(sid * SLT, SLT)],
                  out_hbm.at[pl.ds(cid * ACC_ROWS + sid * SLT, SLT)])


@functools.lru_cache(maxsize=None)
def _deg_sum_kernel():
  mesh = plsc.VectorSubcoreMesh(
      core_axis_name="c", subcore_axis_name="s",
      num_cores=NC, num_subcores=NS)
  return pl.kernel(
      _deg_body, mesh=mesh,
      out_type=jax.ShapeDtypeStruct((NC * ACC_ROWS,), jnp.float32),
      scratch_types=[
          pltpu.VMEM((DCPT, CHUNK), jnp.int32),
          pltpu.VMEM((CHUNK,), jnp.float32),
          pltpu.VMEM((SLT,), jnp.float32),
          pltpu.VMEM_SHARED((ACC_ROWS,), jnp.float32),
          pltpu.SemaphoreType.DMA,
      ],
  )


ROW_BLK = 1000
GRID = N_NODES // ROW_BLK


def _stage_a_body(x_ref, w_ref, d0_ref, d1_ref, hp_ref, dis_ref):
  dis = lax.rsqrt(d0_ref[...] + d1_ref[...] + 1.0)
  h = jnp.dot(x_ref[...], w_ref[...], preferred_element_type=jnp.float32)
  hp_ref[...] = dis * h
  dis_ref[...] = dis


def _stage_a(x, w1, d0, d1):
  return pl.pallas_call(
      _stage_a_body,
      grid=(GRID,),
      in_specs=[
          pl.BlockSpec((ROW_BLK, D), lambda i: (i, 0)),
          pl.BlockSpec((D, D), lambda i: (0, 0)),
          pl.BlockSpec((ROW_BLK, 1), lambda i: (i, 0)),
          pl.BlockSpec((ROW_BLK, 1), lambda i: (i, 0)),
      ],
      out_specs=[
          pl.BlockSpec((ROW_BLK, D), lambda i: (i, 0)),
          pl.BlockSpec((ROW_BLK, 1), lambda i: (i, 0)),
      ],
      out_shape=[
          jax.ShapeDtypeStruct((N_NODES, D), jnp.float32),
          jax.ShapeDtypeStruct((N_NODES, 1), jnp.float32),
      ],
  )(x, w1, d0, d1)


def _stage_b_body(p0_ref, p1_ref, hp_ref, dis_ref, b_ref, w_ref, out_ref):
  dis = dis_ref[...]
  a = dis * (p0_ref[...] + p1_ref[...] + hp_ref[...]) + b_ref[...]
  a = jnp.maximum(a, 0.0)
  out_ref[...] = dis * jnp.dot(a, w_ref[...],
                               preferred_element_type=jnp.float32)


def _stage_b(p0, p1, hp, dis, b1, w2):
  return pl.pallas_call(
      _stage_b_body,
      grid=(GRID,),
      in_specs=[
          pl.BlockSpec((ROW_BLK, D), lambda i: (i, 0)),
          pl.BlockSpec((ROW_BLK, D), lambda i: (i, 0)),
          pl.BlockSpec((ROW_BLK, D), lambda i: (i, 0)),
          pl.BlockSpec((ROW_BLK, 1), lambda i: (i, 0)),
          pl.BlockSpec((1, D), lambda i: (0, 0)),
          pl.BlockSpec((D, D), lambda i: (0, 0)),
      ],
      out_specs=pl.BlockSpec((ROW_BLK, D), lambda i: (i, 0)),
      out_shape=jax.ShapeDtypeStruct((N_NODES, D), jnp.float32),
  )(p0, p1, hp, dis, b1, w2)


def _stage_c_body(p0_ref, p1_ref, hp_ref, dis_ref, b_ref, wfc_ref, bfc_ref,
                  out_ref, acc_ref):
  i = pl.program_id(0)

  @pl.when(i == 0)
  def _():
    acc_ref[...] = jnp.zeros_like(acc_ref)

  a = dis_ref[...] * (p0_ref[...] + p1_ref[...] + hp_ref[...]) + b_ref[...]
  acc_ref[...] += jnp.sum(a, axis=0, keepdims=True)

  @pl.when(i == GRID - 1)
  def _():
    g = acc_ref[...] * (1.0 / N_NODES)
    out_ref[...] = lax.dot_general(
        g, wfc_ref[...], (((1,), (1,)), ((), ())),
        preferred_element_type=jnp.float32) + bfc_ref[...]


def _stage_c(p0, p1, hp, dis, b2, wfc, bfc):
  return pl.pallas_call(
      _stage_c_body,
      grid=(GRID,),
      in_specs=[
          pl.BlockSpec((ROW_BLK, D), lambda i: (i, 0)),
          pl.BlockSpec((ROW_BLK, D), lambda i: (i, 0)),
          pl.BlockSpec((ROW_BLK, D), lambda i: (i, 0)),
          pl.BlockSpec((ROW_BLK, 1), lambda i: (i, 0)),
          pl.BlockSpec((1, D), lambda i: (0, 0)),
          pl.BlockSpec((40, D), lambda i: (0, 0)),
          pl.BlockSpec((1, 40), lambda i: (0, 0)),
      ],
      out_specs=pl.BlockSpec((1, 40), lambda i: (0, 0)),
      out_shape=jax.ShapeDtypeStruct((1, 40), jnp.float32),
      scratch_shapes=[pltpu.VMEM((1, D), jnp.float32)],
  )(p0, p1, hp, dis, b2, wfc, bfc)


def _halves(s):
  return s[:N_NODES], s[ACC_ROWS:ACC_ROWS + N_NODES]


def kernel(x, edge_index, W1, b1, W2, b2, Wfc, bfc):
  row = edge_index[0]
  col = edge_index[1]

  pad1 = E1_SLOTS - E1
  row_sc0 = row[:E0].reshape(NS, CPT0, CHUNK)
  col_sc0 = col[:E0].reshape(NS, CPT0, CHUNK)
  row_sc1 = jnp.concatenate([row[E0:], jnp.zeros((pad1,), jnp.int32)])
  col_sc1 = jnp.concatenate([col[E0:], _pad_dummy(pad1)])
  row_sc1 = row_sc1.reshape(NS, CPT1, CHUNK)
  col_sc1 = col_sc1.reshape(NS, CPT1, CHUNK)
  fill = ((0, 0), (0, CPTM - CPT1), (0, 0))
  row3 = jnp.concatenate(
      [row_sc0, jnp.pad(row_sc1, fill)], axis=0)
  col3 = jnp.concatenate(
      [col_sc0, jnp.pad(col_sc1, fill, constant_values=DUMMY)], axis=0)

  pad_d = DE_PAD - E_EDGES
  col3d = jnp.concatenate([col, _pad_dummy(pad_d)]).reshape(NW, DCPT, CHUNK)

  degp = _deg_sum_kernel()(col3d)
  degp = degp.reshape(-1, 1)
  d0, d1 = _halves(degp)

  hp1, dis = _stage_a(x, W1, d0, d1)
  s1a, s1b = _halves(_seg_sum_kernel()(row3, col3, hp1))
  hp2 = _stage_b(s1a, s1b, hp1, dis, b1.reshape(1, D), W2)
  s2a, s2b = _halves(_seg_sum_kernel()(row3, col3, hp2))
  out = _stage_c(s2a, s2b, hp2, dis, b2.reshape(1, D), Wfc,
                 bfc.reshape(1, 40))
  return out
